# baseline (speedup 1.0000x reference)
import functools

import jax
import jax.numpy as jnp
from jax.experimental import pallas as pl


def _body(x_ref, wl_ref, bl_ref, wd_ref, bd_ref,
          logits_ref, probs_ref, deltas_ref, *, inv_hw):
    x = x_ref[...]
    xm = jnp.sum(x, axis=1) * inv_hw
    logits = (
        jnp.dot(xm, wl_ref[...], preferred_element_type=jnp.float32)
        + bl_ref[...]
    )
    logits_ref[...] = logits
    m = jnp.max(logits, axis=-1, keepdims=True)
    e = jnp.exp(logits - m)
    probs_ref[...] = e / jnp.sum(e, axis=-1, keepdims=True)
    deltas_ref[...] = (
        jnp.dot(xm, wd_ref[...], preferred_element_type=jnp.float32)
        + bd_ref[...]
    )


def kernel(pooled_rois, W_logits, b_logits, W_delta, b_delta):
    n, h, w, ch = pooled_rois.shape
    hw = h * w
    ncls = W_logits.shape[1]
    nd = W_delta.shape[1]

    blk = 256
    while n % blk:
        blk //= 2
    grid = (n // blk,)

    x = pooled_rois.reshape(n, hw, ch)
    bl = b_logits.reshape(1, ncls)
    bd = b_delta.reshape(1, nd)

    body = functools.partial(_body, inv_hw=1.0 / hw)
    logits, probs, deltas = pl.pallas_call(
        body,
        grid=grid,
        in_specs=[
            pl.BlockSpec((blk, hw, ch), lambda i: (i, 0, 0)),
            pl.BlockSpec((ch, ncls), lambda i: (0, 0)),
            pl.BlockSpec((1, ncls), lambda i: (0, 0)),
            pl.BlockSpec((ch, nd), lambda i: (0, 0)),
            pl.BlockSpec((1, nd), lambda i: (0, 0)),
        ],
        out_specs=[
            pl.BlockSpec((blk, ncls), lambda i: (i, 0)),
            pl.BlockSpec((blk, ncls), lambda i: (i, 0)),
            pl.BlockSpec((blk, nd), lambda i: (i, 0)),
        ],
        out_shape=[
            jax.ShapeDtypeStruct((n, ncls), jnp.float32),
            jax.ShapeDtypeStruct((n, ncls), jnp.float32),
            jax.ShapeDtypeStruct((n, nd), jnp.float32),
        ],
    )(x, W_logits, bl, W_delta, bd)
    return (logits, probs, deltas)
